# baseline (speedup 1.0000x reference)
import functools

import jax
import jax.numpy as jnp
from jax import lax
from jax.experimental import pallas as pl
from jax.experimental.pallas import tpu as pltpu
from jax.experimental.pallas import tpu_sc as plsc

N = 10000
D_IN = 128
D_HID = 128
N_CLASSES = 7

_NC = 2
_NS = 16
_NW = _NC * _NS
_K = 128

NPAD = 10112
SEG = NPAD // _NS
_ZR = SEG // 4
D2 = 16

_BM = 1264
_G = 8
_SPLIT0 = 0.75


def _fill_rows(ref, nrows, ncols, value):
    v = jnp.full((16,), value, jnp.float32)

    def body(i, _):
        for jj in range(ncols // 16):
            ref[i, pl.ds(jj * 16, 16)] = v
        return 0

    lax.fori_loop(0, nrows, body, 0)


def _make_deg_kernel(nchunk):
    mesh = plsc.VectorSubcoreMesh(core_axis_name="c", subcore_axis_name="s")

    @functools.partial(
        pl.kernel,
        mesh=mesh,
        out_type=jax.ShapeDtypeStruct((_NC, NPAD, 128), jnp.float32),
        scratch_types=[
            pltpu.VMEM((nchunk, 2, _K), jnp.int32),
            pltpu.VMEM((_K, 128), jnp.float32),
            pltpu.VMEM_SHARED((NPAD, 128), jnp.float32),
        ],
    )
    def deg_kernel(ed_hbm, out_hbm, idx_v, ones_v, shared_deg):
        c = lax.axis_index("c")
        s = lax.axis_index("s")
        _fill_rows(ones_v, _K, 128, 0.0)
        for t in range(4):
            pltpu.sync_copy(ones_v, shared_deg.at[pl.ds(s * SEG + t * _K, _K)])
        rem = SEG - 4 * _K
        pltpu.sync_copy(ones_v.at[pl.ds(0, rem)],
                        shared_deg.at[pl.ds(s * SEG + 4 * _K, rem)])
        _fill_rows(ones_v, _K, 128, 1.0)
        plsc.subcore_barrier()
        pltpu.sync_copy(ed_hbm.at[s, pl.ds(c * nchunk, nchunk)], idx_v)

        def body(j, _):
            pltpu.sync_copy(ones_v, shared_deg.at[idx_v.at[j, 0]], add=True)
            return 0

        lax.fori_loop(0, nchunk, body, 0)
        plsc.subcore_barrier()
        pltpu.sync_copy(
            shared_deg.at[pl.ds(s * SEG, SEG)],
            out_hbm.at[c, pl.ds(s * SEG, SEG)],
        )

    return deg_kernel


def _make_agg_kernel(n0, n1, d, group):
    mesh = plsc.VectorSubcoreMesh(core_axis_name="c", subcore_axis_name="s")

    @functools.partial(
        pl.kernel,
        mesh=mesh,
        out_type=jax.ShapeDtypeStruct((_NC, NPAD, d), jnp.float32),
        scratch_types=[
            pltpu.VMEM((group, 2, _K), jnp.int32),
            pltpu.VMEM((_K, d), jnp.float32),
            pltpu.VMEM((_K, d), jnp.float32),
            pltpu.SemaphoreType.DMA,
            pltpu.SemaphoreType.DMA,
            pltpu.VMEM_SHARED((NPAD, d), jnp.float32),
        ],
    )
    def agg_kernel(hp_hbm, ed_hbm, out_hbm,
                   ib, g_a, g_b, sem_a, sem_b, shared_acc):
        c = lax.axis_index("c")
        s = lax.axis_index("s")
        start = c * n0
        ngroups = lax.select(c == 0, n0 // group, n1 // group)
        _fill_rows(g_a, _K, d, 0.0)
        for t in range(SEG // _K):
            pltpu.sync_copy(g_a, shared_acc.at[pl.ds(s * SEG + t * _K, _K)])
        rem = SEG - (SEG // _K) * _K
        if rem:
            pltpu.sync_copy(g_a.at[pl.ds(0, rem)],
                            shared_acc.at[pl.ds(s * SEG + (SEG // _K) * _K, rem)])
        plsc.subcore_barrier()

        bufs = (g_a, g_b)
        sems = (sem_a, sem_b)

        def body(jj, _):
            base = start + jj * group
            pltpu.sync_copy(ed_hbm.at[s, pl.ds(base, group)], ib)
            pend = pltpu.async_copy(hp_hbm.at[ib.at[0, 0]], bufs[0], sems[0])
            for t in range(group):
                p = t % 2
                nxt = None
                if t + 1 < group:
                    nxt = pltpu.async_copy(
                        hp_hbm.at[ib.at[t + 1, 0]], bufs[1 - p], sems[1 - p])
                pend.wait()
                pltpu.sync_copy(bufs[p], shared_acc.at[ib.at[t, 1]], add=True)
                pend = nxt
            return 0

        lax.fori_loop(0, ngroups, body, 0)

        plsc.subcore_barrier()
        pltpu.sync_copy(
            shared_acc.at[pl.ds(s * SEG, SEG)],
            out_hbm.at[c, pl.ds(s * SEG, SEG)],
        )

    return agg_kernel


def _tc12_body(x_ref, w_ref, b_ref, deg_ref, o_ref):
    i = pl.program_id(0)
    lin = lax.dot_general(
        x_ref[...], w_ref[...], (((1,), (1,)), ((), ())),
        preferred_element_type=jnp.float32) + b_ref[...]
    dd = deg_ref[...]
    dtot = dd[0, :, 0:1] + dd[1, :, 0:1] + 1.0
    dis = lax.rsqrt(dtot)
    rows = lax.broadcasted_iota(jnp.int32, (_BM, 1), 0) + i * _BM
    o_ref[...] = jnp.where(rows < N, dis * lin, 0.0)


def _tc3_body(agg_ref, hp1_ref, deg_ref, w2_ref, b2_ref, o_ref):
    i = pl.program_id(0)
    dd = deg_ref[...]
    dtot = dd[0, :, 0:1] + dd[1, :, 0:1] + 1.0
    dis = lax.rsqrt(dtot)
    aa = agg_ref[...]
    h = jnp.maximum(dis * (aa[0] + aa[1] + hp1_ref[...]), 0.0)
    lin2 = lax.dot_general(
        h, w2_ref[...], (((1,), (1,)), ((), ())),
        preferred_element_type=jnp.float32) + b2_ref[...]
    rows = lax.broadcasted_iota(jnp.int32, (_BM, 1), 0) + i * _BM
    hp2 = jnp.where(rows < N, dis * lin2, 0.0)
    o_ref[...] = jnp.concatenate(
        [hp2, jnp.zeros((_BM, D_HID - D2), jnp.float32)], axis=1)


def _tc4_body(agg_ref, hp2_ref, deg_ref, o_ref):
    dd = deg_ref[...]
    dtot = dd[0, :, 0:1] + dd[1, :, 0:1] + 1.0
    dis = lax.rsqrt(dtot)
    aa = agg_ref[...]
    o = dis * (aa[0, :, :D2] + aa[1, :, :D2] + hp2_ref[:, :D2])
    colm = lax.broadcasted_iota(jnp.int32, (_BM, D2), 1) < N_CLASSES
    m = jnp.max(jnp.where(colm, o, jnp.float32(-1e30)), axis=1, keepdims=True)
    e = jnp.where(colm, jnp.exp(o - m), 0.0)
    ssum = jnp.sum(e, axis=1, keepdims=True)
    o_ref[...] = (o - m) - jnp.log(ssum)


def _row_block(d):
    return pl.BlockSpec((_BM, d), lambda i: (i, 0))


def _pair_block(d):
    return pl.BlockSpec((2, _BM, d), lambda i: (0, i, 0))


def _full_block(shape):
    return pl.BlockSpec(shape, lambda i: tuple(0 for _ in shape))


@jax.jit
def kernel(x, edge_index, W1, b1, W2, b2):
    e = edge_index.shape[1]
    epw = -(-e // _NW)
    nchunk = -(-epw // _K)
    nchunk = -(-nchunk // _G) * _G
    ct = 2 * nchunk
    n0 = (int(ct * _SPLIT0) // _G) * _G
    n1 = ct - n0
    epad = ct * _K * _NS

    row = jnp.full((epad,), N, jnp.int32).at[:e].set(edge_index[0])
    col = jnp.full((epad,), N, jnp.int32).at[:e].set(edge_index[1])
    ed = jnp.stack([row.reshape(_NS, ct, _K),
                    col.reshape(_NS, ct, _K)], axis=2)
    xp = jnp.zeros((NPAD, D_IN), jnp.float32).at[:N].set(x)
    b1r = b1.reshape(1, D_HID)
    w2p = jnp.zeros((D2, D_HID), jnp.float32).at[:N_CLASSES].set(W2)
    b2p = jnp.zeros((1, D2), jnp.float32).at[0, :N_CLASSES].set(b2)

    grid = (NPAD // _BM,)

    deg = _make_deg_kernel(nchunk)(ed)

    hp1 = pl.pallas_call(
        _tc12_body,
        grid=grid,
        in_specs=[_row_block(D_IN), _full_block((D_HID, D_IN)),
                  _full_block((1, D_HID)), _pair_block(128)],
        out_specs=_row_block(D_HID),
        out_shape=jax.ShapeDtypeStruct((NPAD, D_HID), jnp.float32),
    )(xp, W1, b1r, deg)

    agg1 = _make_agg_kernel(n0, n1, D_HID, _G)(hp1, ed)

    hp2 = pl.pallas_call(
        _tc3_body,
        grid=grid,
        in_specs=[_pair_block(D_HID), _row_block(D_HID), _pair_block(128),
                  _full_block((D2, D_HID)), _full_block((1, D2))],
        out_specs=_row_block(D_HID),
        out_shape=jax.ShapeDtypeStruct((NPAD, D_HID), jnp.float32),
    )(agg1, hp1, deg, w2p, b2p)

    agg2 = _make_agg_kernel(n0, n1, D_HID, _G)(hp2, ed)

    out = pl.pallas_call(
        _tc4_body,
        grid=grid,
        in_specs=[_pair_block(D_HID), _row_block(D_HID), _pair_block(128)],
        out_specs=_row_block(D2),
        out_shape=jax.ShapeDtypeStruct((NPAD, D2), jnp.float32),
    )(agg2, hp2, deg)

    return out[:N, :N_CLASSES]


# speedup vs baseline: 1.0115x; 1.0115x over previous
import functools

import jax
import jax.numpy as jnp
from jax import lax
from jax.experimental import pallas as pl
from jax.experimental.pallas import tpu as pltpu
from jax.experimental.pallas import tpu_sc as plsc

N = 10000
D_IN = 128
D_HID = 128
N_CLASSES = 7

_NC = 2
_NS = 16
_NW = _NC * _NS
_K = 128

NPAD = 10112
SEG = NPAD // _NS
_ZR = SEG // 4
D2 = 16

_BM = 1264
_G = 10
_SPLIT0 = 0.75


def _fill_rows(ref, nrows, ncols, value):
    v = jnp.full((16,), value, jnp.float32)

    def body(i, _):
        for jj in range(ncols // 16):
            ref[i, pl.ds(jj * 16, 16)] = v
        return 0

    lax.fori_loop(0, nrows, body, 0)


def _make_deg_kernel(nchunk):
    mesh = plsc.VectorSubcoreMesh(core_axis_name="c", subcore_axis_name="s")

    @functools.partial(
        pl.kernel,
        mesh=mesh,
        out_type=jax.ShapeDtypeStruct((_NC, NPAD, 128), jnp.float32),
        scratch_types=[
            pltpu.VMEM((nchunk, 2, _K), jnp.int32),
            pltpu.VMEM((_K, 128), jnp.float32),
            pltpu.VMEM_SHARED((NPAD, 128), jnp.float32),
        ],
    )
    def deg_kernel(ed_hbm, out_hbm, idx_v, ones_v, shared_deg):
        c = lax.axis_index("c")
        s = lax.axis_index("s")
        _fill_rows(ones_v, _K, 128, 0.0)
        for t in range(4):
            pltpu.sync_copy(ones_v, shared_deg.at[pl.ds(s * SEG + t * _K, _K)])
        rem = SEG - 4 * _K
        pltpu.sync_copy(ones_v.at[pl.ds(0, rem)],
                        shared_deg.at[pl.ds(s * SEG + 4 * _K, rem)])
        _fill_rows(ones_v, _K, 128, 1.0)
        plsc.subcore_barrier()
        pltpu.sync_copy(ed_hbm.at[s, pl.ds(c * nchunk, nchunk)], idx_v)

        def body(j, _):
            pltpu.sync_copy(ones_v, shared_deg.at[idx_v.at[j, 0]], add=True)
            return 0

        lax.fori_loop(0, nchunk, body, 0)
        plsc.subcore_barrier()
        pltpu.sync_copy(
            shared_deg.at[pl.ds(s * SEG, SEG)],
            out_hbm.at[c, pl.ds(s * SEG, SEG)],
        )

    return deg_kernel


def _make_agg_kernel(n0, n1, d, group):
    mesh = plsc.VectorSubcoreMesh(core_axis_name="c", subcore_axis_name="s")

    @functools.partial(
        pl.kernel,
        mesh=mesh,
        out_type=jax.ShapeDtypeStruct((_NC, NPAD, d), jnp.float32),
        scratch_types=[
            pltpu.VMEM((group, 2, _K), jnp.int32),
            pltpu.VMEM((_K, d), jnp.float32),
            pltpu.VMEM((_K, d), jnp.float32),
            pltpu.SemaphoreType.DMA,
            pltpu.SemaphoreType.DMA,
            pltpu.VMEM_SHARED((NPAD, d), jnp.float32),
        ],
    )
    def agg_kernel(hp_hbm, ed_hbm, out_hbm,
                   ib, g_a, g_b, sem_a, sem_b, shared_acc):
        c = lax.axis_index("c")
        s = lax.axis_index("s")
        start = c * n0
        ngroups = lax.select(c == 0, n0 // group, n1 // group)
        _fill_rows(g_a, _K, d, 0.0)
        for t in range(SEG // _K):
            pltpu.sync_copy(g_a, shared_acc.at[pl.ds(s * SEG + t * _K, _K)])
        rem = SEG - (SEG // _K) * _K
        if rem:
            pltpu.sync_copy(g_a.at[pl.ds(0, rem)],
                            shared_acc.at[pl.ds(s * SEG + (SEG // _K) * _K, rem)])
        plsc.subcore_barrier()

        bufs = (g_a, g_b)
        sems = (sem_a, sem_b)

        def body(jj, _):
            base = start + jj * group
            pltpu.sync_copy(ed_hbm.at[s, pl.ds(base, group)], ib)
            pend = pltpu.async_copy(hp_hbm.at[ib.at[0, 0]], bufs[0], sems[0])
            for t in range(group):
                p = t % 2
                nxt = None
                if t + 1 < group:
                    nxt = pltpu.async_copy(
                        hp_hbm.at[ib.at[t + 1, 0]], bufs[1 - p], sems[1 - p])
                pend.wait()
                pltpu.sync_copy(bufs[p], shared_acc.at[ib.at[t, 1]], add=True)
                pend = nxt
            return 0

        lax.fori_loop(0, ngroups, body, 0)
        plsc.subcore_barrier()
        pltpu.sync_copy(
            shared_acc.at[pl.ds(s * SEG, SEG)],
            out_hbm.at[c, pl.ds(s * SEG, SEG)],
        )

    return agg_kernel


def _tc1_body(x_ref, w_ref, b_ref, o_ref):
    o_ref[...] = lax.dot_general(
        x_ref[...], w_ref[...], (((1,), (1,)), ((), ())),
        preferred_element_type=jnp.float32) + b_ref[...]


def _tc2_body(lin_ref, deg_ref, o_ref):
    i = pl.program_id(0)
    dd = deg_ref[...]
    dtot = dd[0, :, 0:1] + dd[1, :, 0:1] + 1.0
    dis = lax.rsqrt(dtot)
    rows = lax.broadcasted_iota(jnp.int32, (_BM, 1), 0) + i * _BM
    o_ref[...] = jnp.where(rows < N, dis * lin_ref[...], 0.0)


def _tc3_body(agg_ref, hp1_ref, deg_ref, w2_ref, b2_ref, o_ref):
    i = pl.program_id(0)
    dd = deg_ref[...]
    dtot = dd[0, :, 0:1] + dd[1, :, 0:1] + 1.0
    dis = lax.rsqrt(dtot)
    aa = agg_ref[...]
    h = jnp.maximum(dis * (aa[0] + aa[1] + hp1_ref[...]), 0.0)
    lin2 = lax.dot_general(
        h, w2_ref[...], (((1,), (1,)), ((), ())),
        preferred_element_type=jnp.float32) + b2_ref[...]
    rows = lax.broadcasted_iota(jnp.int32, (_BM, 1), 0) + i * _BM
    hp2 = jnp.where(rows < N, dis * lin2, 0.0)
    o_ref[...] = jnp.concatenate(
        [hp2, jnp.zeros((_BM, D_HID - D2), jnp.float32)], axis=1)


def _tc4_body(agg_ref, hp2_ref, deg_ref, o_ref):
    dd = deg_ref[...]
    dtot = dd[0, :, 0:1] + dd[1, :, 0:1] + 1.0
    dis = lax.rsqrt(dtot)
    aa = agg_ref[...]
    o = dis * (aa[0, :, :D2] + aa[1, :, :D2] + hp2_ref[:, :D2])
    colm = lax.broadcasted_iota(jnp.int32, (_BM, D2), 1) < N_CLASSES
    m = jnp.max(jnp.where(colm, o, jnp.float32(-1e30)), axis=1, keepdims=True)
    e = jnp.where(colm, jnp.exp(o - m), 0.0)
    ssum = jnp.sum(e, axis=1, keepdims=True)
    o_ref[...] = (o - m) - jnp.log(ssum)


def _row_block(d):
    return pl.BlockSpec((_BM, d), lambda i: (i, 0))


def _pair_block(d):
    return pl.BlockSpec((2, _BM, d), lambda i: (0, i, 0))


def _full_block(shape):
    return pl.BlockSpec(shape, lambda i: tuple(0 for _ in shape))


@jax.jit
def kernel(x, edge_index, W1, b1, W2, b2):
    e = edge_index.shape[1]
    epw = -(-e // _NW)
    nchunk = -(-epw // _K)
    nchunk = -(-nchunk // _G) * _G
    ct = 2 * nchunk
    n0 = (int(ct * _SPLIT0) // _G) * _G
    n1 = ct - n0
    epad = ct * _K * _NS

    row = jnp.full((epad,), N, jnp.int32).at[:e].set(edge_index[0])
    col = jnp.full((epad,), N, jnp.int32).at[:e].set(edge_index[1])
    ed = jnp.stack([row.reshape(_NS, ct, _K),
                    col.reshape(_NS, ct, _K)], axis=2)
    xp = jnp.zeros((NPAD, D_IN), jnp.float32).at[:N].set(x)
    b1r = b1.reshape(1, D_HID)
    w2p = jnp.zeros((D2, D_HID), jnp.float32).at[:N_CLASSES].set(W2)
    b2p = jnp.zeros((1, D2), jnp.float32).at[0, :N_CLASSES].set(b2)

    grid = (NPAD // _BM,)

    deg = _make_deg_kernel(nchunk)(ed)

    lin1 = pl.pallas_call(
        _tc1_body,
        grid=grid,
        in_specs=[_row_block(D_IN), _full_block((D_HID, D_IN)),
                  _full_block((1, D_HID))],
        out_specs=_row_block(D_HID),
        out_shape=jax.ShapeDtypeStruct((NPAD, D_HID), jnp.float32),
    )(xp, W1, b1r)

    hp1 = pl.pallas_call(
        _tc2_body,
        grid=grid,
        in_specs=[_row_block(D_HID), _pair_block(128)],
        out_specs=_row_block(D_HID),
        out_shape=jax.ShapeDtypeStruct((NPAD, D_HID), jnp.float32),
    )(lin1, deg)

    agg1 = _make_agg_kernel(n0, n1, D_HID, _G)(hp1, ed)

    hp2 = pl.pallas_call(
        _tc3_body,
        grid=grid,
        in_specs=[_pair_block(D_HID), _row_block(D_HID), _pair_block(128),
                  _full_block((D2, D_HID)), _full_block((1, D2))],
        out_specs=_row_block(D_HID),
        out_shape=jax.ShapeDtypeStruct((NPAD, D_HID), jnp.float32),
    )(agg1, hp1, deg, w2p, b2p)

    agg2 = _make_agg_kernel(n0, n1, D_HID, _G)(hp2, ed)

    out = pl.pallas_call(
        _tc4_body,
        grid=grid,
        in_specs=[_pair_block(D_HID), _row_block(D_HID), _pair_block(128)],
        out_specs=_row_block(D2),
        out_shape=jax.ShapeDtypeStruct((NPAD, D2), jnp.float32),
    )(agg2, hp2, deg)

    return out[:N, :N_CLASSES]
